# baseline (speedup 1.0000x reference)
import functools

import jax
import jax.numpy as jnp
from jax import lax
from jax.experimental import pallas as pl
from jax.experimental.pallas import tpu as pltpu
import jax.experimental.pallas.tpu_sc as plsc

B, T, C, Q, V, D = 1024, 50, 2, 8, 8192, 64
NC, NS = 2, 16
NW = NC * NS
N = B * T * C * Q
RPW = N // NW
CH = 512
GSUB = 128
NSPLIT = 2


def _expand_body(tab_ref, ch_ref, out_ref):
    q = pl.program_id(0)
    t = tab_ref[0]
    b0 = ch_ref[pl.ds(q, 1), :]
    b1 = ch_ref[pl.ds(Q + q, 1), :]
    out_ref[...] = jnp.concatenate([t + b0, t + b1], axis=1)


def _expand_table(tables, channel_emb):
    return pl.pallas_call(
        _expand_body,
        grid=(Q,),
        in_specs=[
            pl.BlockSpec((1, V, D), lambda q: (q, 0, 0)),
            pl.BlockSpec((C * Q, D), lambda q: (0, 0)),
        ],
        out_specs=pl.BlockSpec((V, 2 * D), lambda q: (q, 0)),
        out_shape=jax.ShapeDtypeStruct((Q * V, 2 * D), jnp.float32),
    )(tables, channel_emb.reshape(C * Q, D))


def _gather_body(codes_hbm, exp_hbm, out_hbm, idx_a, idx_b, rows_a, rows_b,
                 sem_a, sem_b, *, ch, nchunk, sub):
    CH, NCHUNK, SUB = ch, nchunk, sub
    wid = lax.axis_index("s") * NC + lax.axis_index("c")
    base = wid * (NCHUNK * CH)
    lane = lax.iota(jnp.int32, 16)
    offs = ((lane & (Q - 1)) << 14) + (lane >> 3)

    def fire(g, idx_v, rows_v, sem):
        row0 = pl.multiple_of(base + g * CH, CH)
        pltpu.sync_copy(
            codes_hbm.at[pl.ds(pl.multiple_of(row0 // GSUB, SUB), SUB)], idx_v
        )
        for i in range(SUB):
            for j in range(GSUB // 16):
                sl = pl.ds(j * 16, 16)
                idx_v[i, sl] = idx_v[i, sl] * 2 + offs
        for i in range(SUB):
            pltpu.async_copy(
                exp_hbm.at[idx_v.at[i]], rows_v.at[pl.ds(i * GSUB, GSUB)], sem
            )

    def drain(idx_v, rows_v, sem):
        for i in range(SUB):
            pltpu.make_async_copy(
                exp_hbm.at[idx_v.at[i]], rows_v.at[pl.ds(i * GSUB, GSUB)], sem
            ).wait()

    def write(g, rows_v):
        row0 = pl.multiple_of(base + g * CH, CH)
        pltpu.sync_copy(rows_v, out_hbm.at[pl.ds(row0, CH)])

    fire(0, idx_a, rows_a, sem_a)

    @pl.loop(0, NCHUNK, step=2)
    def _pair(g):
        fire(g + 1, idx_b, rows_b, sem_b)
        drain(idx_a, rows_a, sem_a)
        write(g, rows_a)
        @pl.when(g + 2 < NCHUNK)
        def _():
            fire(g + 2, idx_a, rows_a, sem_a)

        drain(idx_b, rows_b, sem_b)
        write(g + 1, rows_b)


@functools.cache
def _make_gather(nrows):
    rpw = nrows // NW
    ch = CH if rpw % (2 * CH) == 0 else CH // 2
    nchunk = rpw // ch
    sub = ch // GSUB
    body = functools.partial(_gather_body, ch=ch, nchunk=nchunk, sub=sub)
    return pl.kernel(
        body,
        out_type=jax.ShapeDtypeStruct((nrows, D), jnp.float32),
        mesh=plsc.VectorSubcoreMesh(
            core_axis_name="c", subcore_axis_name="s", num_cores=NC, num_subcores=NS
        ),
        scratch_types=[
            pltpu.VMEM((sub, GSUB), jnp.int32),
            pltpu.VMEM((sub, GSUB), jnp.int32),
            pltpu.VMEM((ch, D), jnp.float32),
            pltpu.VMEM((ch, D), jnp.float32),
            pltpu.SemaphoreType.DMA,
            pltpu.SemaphoreType.DMA,
        ],
        compiler_params=pltpu.CompilerParams(use_tc_tiling_on_sc=False),
    )


def kernel(codes, tables, channel_emb):
    exp = _expand_table(tables, channel_emb).reshape(C * Q * V, D)
    nrows = N // NSPLIT
    _gather = _make_gather(nrows)
    bsplit = B // NSPLIT
    parts = []
    for s in range(NSPLIT):
        codes_s = lax.slice_in_dim(codes, s * bsplit, (s + 1) * bsplit, axis=0)
        codes2 = codes_s.astype(jnp.int32).reshape(nrows // GSUB, GSUB)
        if parts:
            codes2, _ = lax.optimization_barrier((codes2, parts[-1]))
        parts.append(_gather(codes2, exp).reshape(bsplit, T, C * Q * D))
    return jnp.concatenate(parts, axis=0)


# speedup vs baseline: 1.1583x; 1.1583x over previous
import functools

import jax
import jax.numpy as jnp
from jax import lax
from jax.experimental import pallas as pl
from jax.experimental.pallas import tpu as pltpu
import jax.experimental.pallas.tpu_sc as plsc

B, T, C, Q, V, D = 1024, 50, 2, 8, 8192, 64
NC, NS = 2, 16
NW = NC * NS
N = B * T * C * Q
RPW = N // NW
CH = 512
GSUB = 128
NSPLIT = 1


def _expand_body(tab_ref, ch_ref, out_ref):
    q = pl.program_id(0)
    t = tab_ref[0]
    b0 = ch_ref[pl.ds(q, 1), :]
    b1 = ch_ref[pl.ds(Q + q, 1), :]
    out_ref[...] = jnp.concatenate([t + b0, t + b1], axis=1)


def _expand_table(tables, channel_emb):
    return pl.pallas_call(
        _expand_body,
        grid=(Q,),
        in_specs=[
            pl.BlockSpec((1, V, D), lambda q: (q, 0, 0)),
            pl.BlockSpec((C * Q, D), lambda q: (0, 0)),
        ],
        out_specs=pl.BlockSpec((V, 2 * D), lambda q: (q, 0)),
        out_shape=jax.ShapeDtypeStruct((Q * V, 2 * D), jnp.float32),
    )(tables, channel_emb.reshape(C * Q, D))


def _gather_body(codes_hbm, exp_hbm, out_hbm, idx_a, idx_b, rows_a, rows_b,
                 sem_a, sem_b, *, ch, nchunk, sub):
    CH, NCHUNK, SUB = ch, nchunk, sub
    wid = lax.axis_index("s") * NC + lax.axis_index("c")
    base = wid * (NCHUNK * CH)
    lane = lax.iota(jnp.int32, 16)
    offs = ((lane & (Q - 1)) << 14) + (lane >> 3)

    def fire(g, idx_v, rows_v, sem):
        row0 = pl.multiple_of(base + g * CH, CH)
        pltpu.sync_copy(
            codes_hbm.at[pl.ds(pl.multiple_of(row0 // GSUB, SUB), SUB)], idx_v
        )
        for i in range(SUB):
            for j in range(GSUB // 16):
                sl = pl.ds(j * 16, 16)
                idx_v[i, sl] = idx_v[i, sl] * 2 + offs
        for i in range(SUB):
            pltpu.async_copy(
                exp_hbm.at[idx_v.at[i]], rows_v.at[pl.ds(i * GSUB, GSUB)], sem
            )

    def drain(idx_v, rows_v, sem):
        for i in range(SUB):
            pltpu.make_async_copy(
                exp_hbm.at[idx_v.at[i]], rows_v.at[pl.ds(i * GSUB, GSUB)], sem
            ).wait()

    def write(g, rows_v):
        row0 = pl.multiple_of(base + g * CH, CH)
        pltpu.sync_copy(rows_v, out_hbm.at[pl.ds(row0, CH)])

    fire(0, idx_a, rows_a, sem_a)

    @pl.loop(0, NCHUNK, step=2)
    def _pair(g):
        fire(g + 1, idx_b, rows_b, sem_b)
        drain(idx_a, rows_a, sem_a)
        write(g, rows_a)
        @pl.when(g + 2 < NCHUNK)
        def _():
            fire(g + 2, idx_a, rows_a, sem_a)

        drain(idx_b, rows_b, sem_b)
        write(g + 1, rows_b)


@functools.cache
def _make_gather(nrows):
    rpw = nrows // NW
    ch = CH if rpw % (2 * CH) == 0 else CH // 2
    nchunk = rpw // ch
    sub = ch // GSUB
    body = functools.partial(_gather_body, ch=ch, nchunk=nchunk, sub=sub)
    return pl.kernel(
        body,
        out_type=jax.ShapeDtypeStruct((nrows, D), jnp.float32),
        mesh=plsc.VectorSubcoreMesh(
            core_axis_name="c", subcore_axis_name="s", num_cores=NC, num_subcores=NS
        ),
        scratch_types=[
            pltpu.VMEM((sub, GSUB), jnp.int32),
            pltpu.VMEM((sub, GSUB), jnp.int32),
            pltpu.VMEM((ch, D), jnp.float32),
            pltpu.VMEM((ch, D), jnp.float32),
            pltpu.SemaphoreType.DMA,
            pltpu.SemaphoreType.DMA,
        ],
        compiler_params=pltpu.CompilerParams(use_tc_tiling_on_sc=False),
    )


def kernel(codes, tables, channel_emb):
    exp = _expand_table(tables, channel_emb).reshape(C * Q * V, D)
    nrows = N // NSPLIT
    _gather = _make_gather(nrows)
    bsplit = B // NSPLIT
    parts = []
    for s in range(NSPLIT):
        codes_s = lax.slice_in_dim(codes, s * bsplit, (s + 1) * bsplit, axis=0)
        codes2 = codes_s.astype(jnp.int32).reshape(nrows // GSUB, GSUB)
        if parts:
            codes2, _ = lax.optimization_barrier((codes2, parts[-1]))
        parts.append(_gather(codes2, exp).reshape(bsplit, T, C * Q * D))
    return jnp.concatenate(parts, axis=0)


# speedup vs baseline: 1.1625x; 1.0036x over previous
import functools

import jax
import jax.numpy as jnp
from jax import lax
from jax.experimental import pallas as pl
from jax.experimental.pallas import tpu as pltpu
import jax.experimental.pallas.tpu_sc as plsc

B, T, C, Q, V, D = 1024, 50, 2, 8, 8192, 64
NC, NS = 2, 16
NW = NC * NS
N = B * T * C * Q
RPW = N // NW
CH = 640
GSUB = 128
NSPLIT = 1


def _expand_body(tab_ref, ch_ref, out_ref):
    q = pl.program_id(0)
    t = tab_ref[0]
    b0 = ch_ref[pl.ds(q, 1), :]
    b1 = ch_ref[pl.ds(Q + q, 1), :]
    out_ref[...] = jnp.concatenate([t + b0, t + b1], axis=1)


def _expand_table(tables, channel_emb):
    return pl.pallas_call(
        _expand_body,
        grid=(Q,),
        in_specs=[
            pl.BlockSpec((1, V, D), lambda q: (q, 0, 0)),
            pl.BlockSpec((C * Q, D), lambda q: (0, 0)),
        ],
        out_specs=pl.BlockSpec((V, 2 * D), lambda q: (q, 0)),
        out_shape=jax.ShapeDtypeStruct((Q * V, 2 * D), jnp.float32),
    )(tables, channel_emb.reshape(C * Q, D))


def _gather_body(codes_hbm, exp_hbm, out_hbm, idx_a, idx_b, rows_a, rows_b,
                 sem_a, sem_b, *, ch, nchunk, sub):
    CH, NCHUNK, SUB = ch, nchunk, sub
    wid = lax.axis_index("s") * NC + lax.axis_index("c")
    base = wid * (NCHUNK * CH)
    lane = lax.iota(jnp.int32, 16)
    offs = ((lane & (Q - 1)) << 14) + (lane >> 3)

    def fire(g, idx_v, rows_v, sem):
        row0 = pl.multiple_of(base + g * CH, CH)
        pltpu.sync_copy(
            codes_hbm.at[pl.ds(pl.multiple_of(row0 // GSUB, SUB), SUB)], idx_v
        )
        for i in range(SUB):
            for j in range(GSUB // 16):
                sl = pl.ds(j * 16, 16)
                idx_v[i, sl] = idx_v[i, sl] * 2 + offs
        for i in range(SUB):
            pltpu.async_copy(
                exp_hbm.at[idx_v.at[i]], rows_v.at[pl.ds(i * GSUB, GSUB)], sem
            )

    def drain(idx_v, rows_v, sem):
        for i in range(SUB):
            pltpu.make_async_copy(
                exp_hbm.at[idx_v.at[i]], rows_v.at[pl.ds(i * GSUB, GSUB)], sem
            ).wait()

    def write(g, rows_v):
        row0 = pl.multiple_of(base + g * CH, CH)
        pltpu.sync_copy(rows_v, out_hbm.at[pl.ds(row0, CH)])

    fire(0, idx_a, rows_a, sem_a)

    @pl.loop(0, NCHUNK, step=2)
    def _pair(g):
        fire(g + 1, idx_b, rows_b, sem_b)
        drain(idx_a, rows_a, sem_a)
        write(g, rows_a)
        @pl.when(g + 2 < NCHUNK)
        def _():
            fire(g + 2, idx_a, rows_a, sem_a)

        drain(idx_b, rows_b, sem_b)
        write(g + 1, rows_b)


@functools.cache
def _make_gather(nrows):
    rpw = nrows // NW
    ch = CH if rpw % (2 * CH) == 0 else CH // 2
    nchunk = rpw // ch
    sub = ch // GSUB
    body = functools.partial(_gather_body, ch=ch, nchunk=nchunk, sub=sub)
    return pl.kernel(
        body,
        out_type=jax.ShapeDtypeStruct((nrows, D), jnp.float32),
        mesh=plsc.VectorSubcoreMesh(
            core_axis_name="c", subcore_axis_name="s", num_cores=NC, num_subcores=NS
        ),
        scratch_types=[
            pltpu.VMEM((sub, GSUB), jnp.int32),
            pltpu.VMEM((sub, GSUB), jnp.int32),
            pltpu.VMEM((ch, D), jnp.float32),
            pltpu.VMEM((ch, D), jnp.float32),
            pltpu.SemaphoreType.DMA,
            pltpu.SemaphoreType.DMA,
        ],
        compiler_params=pltpu.CompilerParams(use_tc_tiling_on_sc=False),
    )


def kernel(codes, tables, channel_emb):
    exp = _expand_table(tables, channel_emb).reshape(C * Q * V, D)
    nrows = N // NSPLIT
    _gather = _make_gather(nrows)
    bsplit = B // NSPLIT
    parts = []
    for s in range(NSPLIT):
        codes_s = lax.slice_in_dim(codes, s * bsplit, (s + 1) * bsplit, axis=0)
        codes2 = codes_s.astype(jnp.int32).reshape(nrows // GSUB, GSUB)
        if parts:
            codes2, _ = lax.optimization_barrier((codes2, parts[-1]))
        parts.append(_gather(codes2, exp).reshape(bsplit, T, C * Q * D))
    return jnp.concatenate(parts, axis=0)
